# Initial kernel scaffold; baseline (speedup 1.0000x reference)
import jax
import jax.numpy as jnp
from jax.experimental import pallas as pl


def kernel(h0, h1, h2, nei0, nei1, att0, att1, fc_W, fc_b, att_inter):
    raise NotImplementedError("write your pallas kernel here")


# speedup vs baseline: 2.3182x; 2.3182x over previous
import functools

import jax
import jax.numpy as jnp
from jax import lax
from jax.experimental import pallas as pl
from jax.experimental.pallas import tpu as pltpu
from jax.experimental.pallas import tpu_sc as plsc

_N = 10000
_D = 128
_S0 = 16
_S1 = 32
_TAU = 0.5
_ALPHA = 0.5

_NW = 32
_NPAD = 10240
_TB = _NPAD // _NW
_NC = 2


def _proj_body(h0_ref, h1_ref, h2_ref, att0_ref, att1_ref, o_ref):
    a0 = att0_ref[...]
    a1 = att1_ref[...]
    ar = jnp.concatenate([a0[:, :_D], a1[:, :_D]], axis=0)
    dn = (((1,), (1,)), ((), ()))
    p01 = lax.dot_general(ar, h0_ref[...], dn,
                          preferred_element_type=jnp.float32)
    p2 = lax.dot_general(a0[:, _D:], h1_ref[...], dn,
                         preferred_element_type=jnp.float32)
    p3 = lax.dot_general(a1[:, _D:], h2_ref[...], dn,
                         preferred_element_type=jnp.float32)
    o_ref[...] = jnp.concatenate(
        [p01, p2, p3, jnp.zeros_like(p01), p2, p3], axis=0)


def _proj(h0p, h1p, h2p, att0, att1):
    B = 1024
    return pl.pallas_call(
        _proj_body,
        grid=(_NPAD // B,),
        in_specs=[
            pl.BlockSpec((B, _D), lambda i: (i, 0)),
            pl.BlockSpec((B, _D), lambda i: (i, 0)),
            pl.BlockSpec((B, _D), lambda i: (i, 0)),
            pl.BlockSpec((1, 2 * _D), lambda i: (0, 0)),
            pl.BlockSpec((1, 2 * _D), lambda i: (0, 0)),
        ],
        out_specs=pl.BlockSpec((8, B), lambda i: (0, i)),
        out_shape=jax.ShapeDtypeStruct((8, _NPAD), jnp.float32),
    )(h0p, h1p, h2p, att0, att1)


def _sc_agg(h1, h2, nei0p, nei1p, P):
    mesh = plsc.VectorSubcoreMesh(core_axis_name="c", subcore_axis_name="s")

    @functools.partial(
        pl.kernel,
        out_type=[jax.ShapeDtypeStruct((_NPAD, _D), jnp.float32),
                  jax.ShapeDtypeStruct((_NPAD, _D), jnp.float32)],
        mesh=mesh,
        compiler_params=pltpu.CompilerParams(needs_layout_passes=False,
                                             use_tc_tiling_on_sc=False),
        scratch_types=[
            pltpu.VMEM((_NPAD,), jnp.float32),
            pltpu.VMEM((_NPAD,), jnp.float32),
            pltpu.VMEM((_NPAD,), jnp.float32),
            pltpu.VMEM((_NPAD,), jnp.float32),
            pltpu.VMEM((_TB, _S0), jnp.int32),
            pltpu.VMEM((_TB, _S1), jnp.int32),
            pltpu.VMEM((_S1, _D), jnp.float32),
            pltpu.VMEM((_TB, _D), jnp.float32),
            pltpu.SemaphoreType.DMA,
        ],
    )
    def body(h1_hbm, h2_hbm, nei0_hbm, nei1_hbm, p_hbm, e0_hbm, e1_hbm,
             p0_ts, p1_ts, p2_ts, p3_ts, nei0_ts, nei1_ts, rows_v, e_buf,
             sem):
        wid = lax.axis_index("s") * _NC + lax.axis_index("c")
        base = wid * _TB
        pltpu.sync_copy(p_hbm.at[0], p0_ts)
        pltpu.sync_copy(p_hbm.at[1], p1_ts)
        pltpu.sync_copy(p_hbm.at[2], p2_ts)
        pltpu.sync_copy(p_hbm.at[3], p3_ts)
        pltpu.sync_copy(nei0_hbm.at[pl.ds(base, _TB)], nei0_ts)
        pltpu.sync_copy(nei1_hbm.at[pl.ds(base, _TB)], nei1_ts)

        def run_view(h_hbm, nei_ts, s_count, pr_ts, pv_ts, e_hbm):
            nvec = s_count // 16

            def target(i, carry):
                gi = base + i
                idxs = [nei_ts[i, pl.ds(16 * v, 16)] for v in range(nvec)]
                cps = [pltpu.async_copy(h_hbm.at[idxs[v]],
                                        rows_v.at[pl.ds(16 * v, 16)], sem)
                       for v in range(nvec)]
                pr = plsc.load_gather(pr_ts, [jnp.full((16,), gi, jnp.int32)])
                lgs = []
                for v in range(nvec):
                    pv = plsc.load_gather(pv_ts, [idxs[v]])
                    lg = pr + pv
                    lgs.append(jnp.where(lg >= 0.0, lg, 0.01 * lg))
                m = jnp.max(lgs[0])
                for v in range(1, nvec):
                    m = jnp.maximum(m, jnp.max(lgs[v]))
                exs = [jnp.exp(lg - m) for lg in lgs]
                ssum = jnp.sum(exs[0])
                for v in range(1, nvec):
                    ssum = ssum + jnp.sum(exs[v])
                denom = jnp.full((16,), ssum, jnp.float32)
                ws_all = [exs[v] / denom for v in range(nvec)]
                for cp in cps:
                    cp.wait()
                iota = lax.iota(jnp.int32, 16)
                accs = [jnp.zeros((16,), jnp.float32)
                        for _ in range(_D // 16)]
                for s_ in range(s_count):
                    wv = ws_all[s_ // 16]
                    ws = jnp.full(
                        (16,),
                        jnp.sum(jnp.where(iota == (s_ % 16), wv, 0.0)),
                        jnp.float32)
                    for dc in range(_D // 16):
                        accs[dc] = accs[dc] + ws * rows_v[s_, pl.ds(16 * dc, 16)]
                for dc in range(_D // 16):
                    a = accs[dc]
                    e_buf[i, pl.ds(16 * dc, 16)] = jnp.where(
                        a > 0.0, a, jnp.exp(a) - 1.0)
                return carry

            lax.fori_loop(0, _TB, target, 0)
            pltpu.sync_copy(e_buf, e_hbm.at[pl.ds(base, _TB)])

        run_view(h1_hbm, nei0_ts, _S0, p0_ts, p2_ts, e0_hbm)
        run_view(h2_hbm, nei1_ts, _S1, p1_ts, p3_ts, e1_hbm)

    return body(h1, h2, nei0p, nei1p, P)


def _prep_body(h0_ref, e0_ref, e1_ref, fcw_ref, fcb_ref,
               h0n_ref, e0n_ref, e1n_ref, sp0_ref, sp1_ref):
    i = pl.program_id(0)

    def nrm(x):
        n = jnp.sqrt(jnp.sum(x * x, axis=1, keepdims=True))
        return x / (n + 1e-8)

    e0 = e0_ref[...]
    e1 = e1_ref[...]
    h0n_ref[...] = nrm(h0_ref[...])
    e0n_ref[...] = nrm(e0)
    e1n_ref[...] = nrm(e1)
    dn = (((1,), (1,)), ((), ()))
    fcw = fcw_ref[...]
    fcb = fcb_ref[...]
    t0 = jnp.tanh(lax.dot_general(e0, fcw, dn,
                                  preferred_element_type=jnp.float32) + fcb)
    t1 = jnp.tanh(lax.dot_general(e1, fcw, dn,
                                  preferred_element_type=jnp.float32) + fcb)

    @pl.when(i == 0)
    def _():
        sp0_ref[...] = jnp.zeros_like(sp0_ref)
        sp1_ref[...] = jnp.zeros_like(sp1_ref)

    sp0_ref[...] += jnp.sum(t0, axis=0, keepdims=True)
    sp1_ref[...] += jnp.sum(t1, axis=0, keepdims=True)


def _prep(h0, e0, e1, fc_W, fc_b2):
    B = 1000
    row = lambda i: (i, 0)
    fixed = lambda i: (0, 0)
    return pl.pallas_call(
        _prep_body,
        grid=(_N // B,),
        in_specs=[
            pl.BlockSpec((B, _D), row),
            pl.BlockSpec((B, _D), row),
            pl.BlockSpec((B, _D), row),
            pl.BlockSpec((_D, _D), fixed),
            pl.BlockSpec((1, _D), fixed),
        ],
        out_specs=[
            pl.BlockSpec((B, _D), row),
            pl.BlockSpec((B, _D), row),
            pl.BlockSpec((B, _D), row),
            pl.BlockSpec((1, _D), fixed),
            pl.BlockSpec((1, _D), fixed),
        ],
        out_shape=[
            jax.ShapeDtypeStruct((_N, _D), jnp.float32),
            jax.ShapeDtypeStruct((_N, _D), jnp.float32),
            jax.ShapeDtypeStruct((_N, _D), jnp.float32),
            jax.ShapeDtypeStruct((1, _D), jnp.float32),
            jax.ShapeDtypeStruct((1, _D), jnp.float32),
        ],
    )(h0, e0, e1, fc_W, fc_b2)


def _flash_body(h0n_ref, e0n_ref, e1n_ref, e0_ref, e1_ref,
                e0nf_ref, e1nf_ref, sp0_ref, sp1_ref, ai_ref,
                z_ref, la0_ref, la1_ref, la2_ref):
    i = pl.program_id(0)
    inv_tau = 1.0 / _TAU
    dn = (((1,), (1,)), ((), ()))
    h0n = h0n_ref[...]
    e0n = e0n_ref[...]
    e1n = e1n_ref[...]
    e0nf = e0nf_ref[...]
    e1nf = e1nf_ref[...]

    def ntx_part(rows, colsf, diag_rows):
        s = lax.dot_general(rows, colsf, dn,
                            preferred_element_type=jnp.float32) * inv_tau
        lse = jnp.log(jnp.sum(jnp.exp(s), axis=1, keepdims=True))
        d = jnp.sum(rows * diag_rows, axis=1, keepdims=True) * inv_tau
        return jnp.sum(lse - d)

    p0 = ntx_part(h0n, e0nf, e0n)
    p1 = ntx_part(h0n, e1nf, e1n)
    p2 = ntx_part(e0n, e1nf, e1n)

    @pl.when(i == 0)
    def _():
        la0_ref[...] = jnp.zeros_like(la0_ref)
        la1_ref[...] = jnp.zeros_like(la1_ref)
        la2_ref[...] = jnp.zeros_like(la2_ref)

    la0_ref[...] += jnp.full((1, _D), p0, jnp.float32)
    la1_ref[...] += jnp.full((1, _D), p1, jnp.float32)
    la2_ref[...] += jnp.full((1, _D), p2, jnp.float32)

    ai = ai_ref[...]
    b0 = jnp.sum(ai * sp0_ref[...]) * (1.0 / _N)
    b1 = jnp.sum(ai * sp1_ref[...]) * (1.0 / _N)
    t = jnp.exp(jnp.full((1, _D), b1 - b0, jnp.float32))
    beta0 = 1.0 / (1.0 + t)
    e0b = e0_ref[...]
    e1b = e1_ref[...]
    z_ref[...] = e1b + beta0 * (e0b - e1b)


def _flash(h0n, e0n, e1n, e0, e1, sp0s, sp1s, att_inter):
    B = 200
    row = lambda i: (i, 0)
    fixed = lambda i: (0, 0)
    return pl.pallas_call(
        _flash_body,
        grid=(_N // B,),
        in_specs=[
            pl.BlockSpec((B, _D), row),
            pl.BlockSpec((B, _D), row),
            pl.BlockSpec((B, _D), row),
            pl.BlockSpec((B, _D), row),
            pl.BlockSpec((B, _D), row),
            pl.BlockSpec((_N, _D), fixed),
            pl.BlockSpec((_N, _D), fixed),
            pl.BlockSpec((1, _D), fixed),
            pl.BlockSpec((1, _D), fixed),
            pl.BlockSpec((1, _D), fixed),
        ],
        out_specs=[
            pl.BlockSpec((B, _D), row),
            pl.BlockSpec((1, _D), fixed),
            pl.BlockSpec((1, _D), fixed),
            pl.BlockSpec((1, _D), fixed),
        ],
        out_shape=[
            jax.ShapeDtypeStruct((_N, _D), jnp.float32),
            jax.ShapeDtypeStruct((1, _D), jnp.float32),
            jax.ShapeDtypeStruct((1, _D), jnp.float32),
            jax.ShapeDtypeStruct((1, _D), jnp.float32),
        ],
    )(h0n, e0n, e1n, e0, e1, e0n, e1n, sp0s, sp1s, att_inter)


def kernel(h0, h1, h2, nei0, nei1, att0, att1, fc_W, fc_b, att_inter):
    pad = _NPAD - _N
    h0p = jnp.pad(h0, ((0, pad), (0, 0)))
    h1p = jnp.pad(h1, ((0, pad), (0, 0)))
    h2p = jnp.pad(h2, ((0, pad), (0, 0)))
    nei0p = jnp.pad(nei0, ((0, pad), (0, 0)))
    nei1p = jnp.pad(nei1, ((0, pad), (0, 0)))

    P = _proj(h0p, h1p, h2p, att0, att1)
    e0p, e1p = _sc_agg(h1, h2, nei0p, nei1p, P)
    e0 = e0p[:_N]
    e1 = e1p[:_N]
    h0n, e0n, e1n, sp0s, sp1s = _prep(h0, e0, e1, fc_W,
                                      fc_b.reshape(1, _D))
    z_mc, la0, la1, la2 = _flash(h0n, e0n, e1n, e0, e1, sp0s, sp1s,
                                 att_inter)
    loss_total = (la0[0, 0] + la1[0, 0] + _ALPHA * la2[0, 0]) / _N
    return (z_mc, loss_total)


# speedup vs baseline: 2.9724x; 1.2822x over previous
import functools

import jax
import jax.numpy as jnp
from jax import lax
from jax.experimental import pallas as pl
from jax.experimental.pallas import tpu as pltpu
from jax.experimental.pallas import tpu_sc as plsc

_N = 10000
_D = 128
_S0 = 16
_S1 = 32
_TAU = 0.5
_ALPHA = 0.5

_NW = 32
_NPAD = 10240
_TB = _NPAD // _NW
_NC = 2


def _proj_body(h0_ref, h1_ref, h2_ref, att0_ref, att1_ref, o_ref):
    a0 = att0_ref[...]
    a1 = att1_ref[...]
    ar = jnp.concatenate([a0[:, :_D], a1[:, :_D]], axis=0)
    dn = (((1,), (1,)), ((), ()))
    p01 = lax.dot_general(ar, h0_ref[...], dn,
                          preferred_element_type=jnp.float32)
    p2 = lax.dot_general(a0[:, _D:], h1_ref[...], dn,
                         preferred_element_type=jnp.float32)
    p3 = lax.dot_general(a1[:, _D:], h2_ref[...], dn,
                         preferred_element_type=jnp.float32)
    o_ref[...] = jnp.concatenate(
        [p01, p2, p3, jnp.zeros_like(p01), p2, p3], axis=0)


def _proj(h0p, h1p, h2p, att0, att1):
    B = 1024
    return pl.pallas_call(
        _proj_body,
        grid=(_NPAD // B,),
        in_specs=[
            pl.BlockSpec((B, _D), lambda i: (i, 0)),
            pl.BlockSpec((B, _D), lambda i: (i, 0)),
            pl.BlockSpec((B, _D), lambda i: (i, 0)),
            pl.BlockSpec((1, 2 * _D), lambda i: (0, 0)),
            pl.BlockSpec((1, 2 * _D), lambda i: (0, 0)),
        ],
        out_specs=pl.BlockSpec((8, B), lambda i: (0, i)),
        out_shape=jax.ShapeDtypeStruct((8, _NPAD), jnp.float32),
    )(h0p, h1p, h2p, att0, att1)


def _sc_agg(h1, h2, nei0p, nei1p, P):
    mesh = plsc.VectorSubcoreMesh(core_axis_name="c", subcore_axis_name="s")

    @functools.partial(
        pl.kernel,
        out_type=[jax.ShapeDtypeStruct((_NPAD, _D), jnp.float32),
                  jax.ShapeDtypeStruct((_NPAD, _D), jnp.float32)],
        mesh=mesh,
        compiler_params=pltpu.CompilerParams(needs_layout_passes=False,
                                             use_tc_tiling_on_sc=False),
        scratch_types=[
            pltpu.VMEM((_NPAD,), jnp.float32),
            pltpu.VMEM((_NPAD,), jnp.float32),
            pltpu.VMEM((_NPAD,), jnp.float32),
            pltpu.VMEM((_NPAD,), jnp.float32),
            pltpu.VMEM((_TB, _S0), jnp.int32),
            pltpu.VMEM((_TB, _S1), jnp.int32),
            pltpu.VMEM((2, _S1, _D), jnp.float32),
            pltpu.VMEM((_TB, _D), jnp.float32),
            pltpu.SemaphoreType.DMA,
            pltpu.SemaphoreType.DMA,
        ],
    )
    def body(h1_hbm, h2_hbm, nei0_hbm, nei1_hbm, p_hbm, e0_hbm, e1_hbm,
             p0_ts, p1_ts, p2_ts, p3_ts, nei0_ts, nei1_ts, rows_v, e_buf,
             sem0, sem1):
        wid = lax.axis_index("s") * _NC + lax.axis_index("c")
        base = wid * _TB
        pltpu.sync_copy(p_hbm.at[0], p0_ts)
        pltpu.sync_copy(p_hbm.at[1], p1_ts)
        pltpu.sync_copy(p_hbm.at[2], p2_ts)
        pltpu.sync_copy(p_hbm.at[3], p3_ts)
        pltpu.sync_copy(nei0_hbm.at[pl.ds(base, _TB)], nei0_ts)
        pltpu.sync_copy(nei1_hbm.at[pl.ds(base, _TB)], nei1_ts)

        def run_view(h_hbm, nei_ts, s_count, pr_ts, pv_ts, e_hbm):
            nvec = s_count // 16
            sems = (sem0, sem1)

            def fire(i, buf):
                for v in range(nvec):
                    idx = nei_ts[i, pl.ds(16 * v, 16)]
                    pltpu.async_copy(h_hbm.at[idx],
                                     rows_v.at[buf, pl.ds(16 * v, 16)],
                                     sems[buf])

            fire(0, 0)

            def target(i, carry):
                gi = base + i
                buf = lax.rem(i, 2)

                @pl.when(jnp.logical_and(i + 1 < _TB, buf == 0))
                def _():
                    fire(i + 1, 1)

                @pl.when(jnp.logical_and(i + 1 < _TB, buf == 1))
                def _():
                    fire(i + 1, 0)

                idxs = [nei_ts[i, pl.ds(16 * v, 16)] for v in range(nvec)]
                pr = plsc.load_gather(pr_ts, [jnp.full((16,), gi, jnp.int32)])
                lgs = []
                for v in range(nvec):
                    pv = plsc.load_gather(pv_ts, [idxs[v]])
                    lg = pr + pv
                    lgs.append(jnp.where(lg >= 0.0, lg, 0.01 * lg))
                m = jnp.max(lgs[0])
                for v in range(1, nvec):
                    m = jnp.maximum(m, jnp.max(lgs[v]))
                exs = [jnp.exp(lg - m) for lg in lgs]
                ssum = jnp.sum(exs[0])
                for v in range(1, nvec):
                    ssum = ssum + jnp.sum(exs[v])
                denom = jnp.full((16,), ssum, jnp.float32)
                ws_all = [exs[v] / denom for v in range(nvec)]

                @pl.when(buf == 0)
                def _():
                    pltpu.make_async_copy(
                        h_hbm.at[pl.ds(0, s_count)],
                        rows_v.at[0, pl.ds(0, s_count)], sem0).wait()

                @pl.when(buf == 1)
                def _():
                    pltpu.make_async_copy(
                        h_hbm.at[pl.ds(0, s_count)],
                        rows_v.at[1, pl.ds(0, s_count)], sem1).wait()

                iota = lax.iota(jnp.int32, 16)
                accs = [jnp.zeros((16,), jnp.float32)
                        for _ in range(_D // 16)]
                for s_ in range(s_count):
                    wv = ws_all[s_ // 16]
                    ws = jnp.full(
                        (16,),
                        jnp.sum(jnp.where(iota == (s_ % 16), wv, 0.0)),
                        jnp.float32)
                    for dc in range(_D // 16):
                        accs[dc] = accs[dc] + ws * rows_v[buf, s_,
                                                          pl.ds(16 * dc, 16)]
                for dc in range(_D // 16):
                    a = accs[dc]
                    e_buf[i, pl.ds(16 * dc, 16)] = jnp.where(
                        a > 0.0, a, jnp.exp(a) - 1.0)
                return carry

            lax.fori_loop(0, _TB, target, 0)
            pltpu.sync_copy(e_buf, e_hbm.at[pl.ds(base, _TB)])

        run_view(h1_hbm, nei0_ts, _S0, p0_ts, p2_ts, e0_hbm)
        run_view(h2_hbm, nei1_ts, _S1, p1_ts, p3_ts, e1_hbm)

    return body(h1, h2, nei0p, nei1p, P)


def _prep_body(h0_ref, e0_ref, e1_ref, fcw_ref, fcb_ref,
               h0n_ref, e0n_ref, e1n_ref, sp0_ref, sp1_ref):
    i = pl.program_id(0)

    def nrm(x):
        n = jnp.sqrt(jnp.sum(x * x, axis=1, keepdims=True))
        return x / (n + 1e-8)

    e0 = e0_ref[...]
    e1 = e1_ref[...]
    h0n_ref[...] = nrm(h0_ref[...])
    e0n_ref[...] = nrm(e0)
    e1n_ref[...] = nrm(e1)
    dn = (((1,), (1,)), ((), ()))
    fcw = fcw_ref[...]
    fcb = fcb_ref[...]
    t0 = jnp.tanh(lax.dot_general(e0, fcw, dn,
                                  preferred_element_type=jnp.float32) + fcb)
    t1 = jnp.tanh(lax.dot_general(e1, fcw, dn,
                                  preferred_element_type=jnp.float32) + fcb)

    @pl.when(i == 0)
    def _():
        sp0_ref[...] = jnp.zeros_like(sp0_ref)
        sp1_ref[...] = jnp.zeros_like(sp1_ref)

    sp0_ref[...] += jnp.sum(t0, axis=0, keepdims=True)
    sp1_ref[...] += jnp.sum(t1, axis=0, keepdims=True)


def _prep(h0, e0, e1, fc_W, fc_b2):
    B = 1000
    row = lambda i: (i, 0)
    fixed = lambda i: (0, 0)
    return pl.pallas_call(
        _prep_body,
        grid=(_N // B,),
        in_specs=[
            pl.BlockSpec((B, _D), row),
            pl.BlockSpec((B, _D), row),
            pl.BlockSpec((B, _D), row),
            pl.BlockSpec((_D, _D), fixed),
            pl.BlockSpec((1, _D), fixed),
        ],
        out_specs=[
            pl.BlockSpec((B, _D), row),
            pl.BlockSpec((B, _D), row),
            pl.BlockSpec((B, _D), row),
            pl.BlockSpec((1, _D), fixed),
            pl.BlockSpec((1, _D), fixed),
        ],
        out_shape=[
            jax.ShapeDtypeStruct((_N, _D), jnp.float32),
            jax.ShapeDtypeStruct((_N, _D), jnp.float32),
            jax.ShapeDtypeStruct((_N, _D), jnp.float32),
            jax.ShapeDtypeStruct((1, _D), jnp.float32),
            jax.ShapeDtypeStruct((1, _D), jnp.float32),
        ],
    )(h0, e0, e1, fc_W, fc_b2)


def _flash_body(h0n_ref, e0n_ref, e1n_ref, e0_ref, e1_ref,
                e0nf_ref, e1nf_ref, sp0_ref, sp1_ref, ai_ref,
                z_ref, la0_ref, la1_ref, la2_ref):
    i = pl.program_id(0)
    inv_tau = 1.0 / _TAU
    dn = (((1,), (1,)), ((), ()))
    h0n = h0n_ref[...]
    e0n = e0n_ref[...]
    e1n = e1n_ref[...]
    e0nf = e0nf_ref[...]
    e1nf = e1nf_ref[...]

    def ntx_part(rows, colsf, diag_rows):
        s = lax.dot_general(rows, colsf, dn,
                            preferred_element_type=jnp.float32) * inv_tau
        lse = jnp.log(jnp.sum(jnp.exp(s), axis=1, keepdims=True))
        d = jnp.sum(rows * diag_rows, axis=1, keepdims=True) * inv_tau
        return jnp.sum(lse - d)

    p0 = ntx_part(h0n, e0nf, e0n)
    p1 = ntx_part(h0n, e1nf, e1n)
    p2 = ntx_part(e0n, e1nf, e1n)

    @pl.when(i == 0)
    def _():
        la0_ref[...] = jnp.zeros_like(la0_ref)
        la1_ref[...] = jnp.zeros_like(la1_ref)
        la2_ref[...] = jnp.zeros_like(la2_ref)

    la0_ref[...] += jnp.full((1, _D), p0, jnp.float32)
    la1_ref[...] += jnp.full((1, _D), p1, jnp.float32)
    la2_ref[...] += jnp.full((1, _D), p2, jnp.float32)

    ai = ai_ref[...]
    b0 = jnp.sum(ai * sp0_ref[...]) * (1.0 / _N)
    b1 = jnp.sum(ai * sp1_ref[...]) * (1.0 / _N)
    t = jnp.exp(jnp.full((1, _D), b1 - b0, jnp.float32))
    beta0 = 1.0 / (1.0 + t)
    e0b = e0_ref[...]
    e1b = e1_ref[...]
    z_ref[...] = e1b + beta0 * (e0b - e1b)


def _flash(h0n, e0n, e1n, e0, e1, sp0s, sp1s, att_inter):
    B = 200
    row = lambda i: (i, 0)
    fixed = lambda i: (0, 0)
    return pl.pallas_call(
        _flash_body,
        grid=(_N // B,),
        in_specs=[
            pl.BlockSpec((B, _D), row),
            pl.BlockSpec((B, _D), row),
            pl.BlockSpec((B, _D), row),
            pl.BlockSpec((B, _D), row),
            pl.BlockSpec((B, _D), row),
            pl.BlockSpec((_N, _D), fixed),
            pl.BlockSpec((_N, _D), fixed),
            pl.BlockSpec((1, _D), fixed),
            pl.BlockSpec((1, _D), fixed),
            pl.BlockSpec((1, _D), fixed),
        ],
        out_specs=[
            pl.BlockSpec((B, _D), row),
            pl.BlockSpec((1, _D), fixed),
            pl.BlockSpec((1, _D), fixed),
            pl.BlockSpec((1, _D), fixed),
        ],
        out_shape=[
            jax.ShapeDtypeStruct((_N, _D), jnp.float32),
            jax.ShapeDtypeStruct((1, _D), jnp.float32),
            jax.ShapeDtypeStruct((1, _D), jnp.float32),
            jax.ShapeDtypeStruct((1, _D), jnp.float32),
        ],
    )(h0n, e0n, e1n, e0, e1, e0n, e1n, sp0s, sp1s, att_inter)


def kernel(h0, h1, h2, nei0, nei1, att0, att1, fc_W, fc_b, att_inter):
    pad = _NPAD - _N
    h0p = jnp.pad(h0, ((0, pad), (0, 0)))
    h1p = jnp.pad(h1, ((0, pad), (0, 0)))
    h2p = jnp.pad(h2, ((0, pad), (0, 0)))
    nei0p = jnp.pad(nei0, ((0, pad), (0, 0)))
    nei1p = jnp.pad(nei1, ((0, pad), (0, 0)))

    P = _proj(h0p, h1p, h2p, att0, att1)
    e0p, e1p = _sc_agg(h1, h2, nei0p, nei1p, P)
    e0 = e0p[:_N]
    e1 = e1p[:_N]
    h0n, e0n, e1n, sp0s, sp1s = _prep(h0, e0, e1, fc_W,
                                      fc_b.reshape(1, _D))
    z_mc, la0, la1, la2 = _flash(h0n, e0n, e1n, e0, e1, sp0s, sp1s,
                                 att_inter)
    loss_total = (la0[0, 0] + la1[0, 0] + _ALPHA * la2[0, 0]) / _N
    return (z_mc, loss_total)


# speedup vs baseline: 3.1894x; 1.0730x over previous
import functools

import jax
import jax.numpy as jnp
from jax import lax
from jax.experimental import pallas as pl
from jax.experimental.pallas import tpu as pltpu
from jax.experimental.pallas import tpu_sc as plsc

_N = 10000
_D = 128
_S0 = 16
_S1 = 32
_TAU = 0.5
_ALPHA = 0.5

_NW = 32
_NPAD = 10240
_TB = _NPAD // _NW
_NC = 2


def _proj_body(h0_ref, h1_ref, h2_ref, att0_ref, att1_ref, o_ref):
    a0 = att0_ref[...]
    a1 = att1_ref[...]
    ar = jnp.concatenate([a0[:, :_D], a1[:, :_D]], axis=0)
    dn = (((1,), (1,)), ((), ()))
    p01 = lax.dot_general(ar, h0_ref[...], dn,
                          preferred_element_type=jnp.float32)
    p2 = lax.dot_general(a0[:, _D:], h1_ref[...], dn,
                         preferred_element_type=jnp.float32)
    p3 = lax.dot_general(a1[:, _D:], h2_ref[...], dn,
                         preferred_element_type=jnp.float32)
    o_ref[...] = jnp.concatenate(
        [p01, p2, p3, jnp.zeros_like(p01), p2, p3], axis=0)


def _proj(h0p, h1p, h2p, att0, att1):
    B = 1024
    return pl.pallas_call(
        _proj_body,
        grid=(_NPAD // B,),
        in_specs=[
            pl.BlockSpec((B, _D), lambda i: (i, 0)),
            pl.BlockSpec((B, _D), lambda i: (i, 0)),
            pl.BlockSpec((B, _D), lambda i: (i, 0)),
            pl.BlockSpec((1, 2 * _D), lambda i: (0, 0)),
            pl.BlockSpec((1, 2 * _D), lambda i: (0, 0)),
        ],
        out_specs=pl.BlockSpec((8, B), lambda i: (0, i)),
        out_shape=jax.ShapeDtypeStruct((8, _NPAD), jnp.float32),
    )(h0p, h1p, h2p, att0, att1)


def _sc_agg(h1, h2, nei0p, nei1p, P):
    mesh = plsc.VectorSubcoreMesh(core_axis_name="c", subcore_axis_name="s")

    @functools.partial(
        pl.kernel,
        out_type=[jax.ShapeDtypeStruct((_NPAD, _D), jnp.float32),
                  jax.ShapeDtypeStruct((_NPAD, _D), jnp.float32)],
        mesh=mesh,
        compiler_params=pltpu.CompilerParams(needs_layout_passes=False,
                                             use_tc_tiling_on_sc=False),
        scratch_types=[
            pltpu.VMEM((_NPAD,), jnp.float32),
            pltpu.VMEM((_NPAD,), jnp.float32),
            pltpu.VMEM((_NPAD,), jnp.float32),
            pltpu.VMEM((_NPAD,), jnp.float32),
            pltpu.VMEM((_TB, _S0), jnp.int32),
            pltpu.VMEM((_TB, _S1), jnp.int32),
            pltpu.VMEM((4, _S1, _D), jnp.float32),
            pltpu.VMEM((_TB, _D), jnp.float32),
            pltpu.SemaphoreType.DMA,
            pltpu.SemaphoreType.DMA,
            pltpu.SemaphoreType.DMA,
            pltpu.SemaphoreType.DMA,
        ],
    )
    def body(h1_hbm, h2_hbm, nei0_hbm, nei1_hbm, p_hbm, e0_hbm, e1_hbm,
             p0_ts, p1_ts, p2_ts, p3_ts, nei0_ts, nei1_ts, rows_v, e_buf,
             sem0, sem1, sem2, sem3):
        wid = lax.axis_index("s") * _NC + lax.axis_index("c")
        base = wid * _TB
        pltpu.sync_copy(p_hbm.at[0], p0_ts)
        pltpu.sync_copy(p_hbm.at[1], p1_ts)
        pltpu.sync_copy(p_hbm.at[2], p2_ts)
        pltpu.sync_copy(p_hbm.at[3], p3_ts)
        pltpu.sync_copy(nei0_hbm.at[pl.ds(base, _TB)], nei0_ts)
        pltpu.sync_copy(nei1_hbm.at[pl.ds(base, _TB)], nei1_ts)

        def run_view(h_hbm, nei_ts, s_count, pr_ts, pv_ts, e_hbm):
            nvec = s_count // 16
            sems = (sem0, sem1, sem2, sem3)
            nbuf = 4

            def fire(i, buf):
                for v in range(nvec):
                    idx = nei_ts[i, pl.ds(16 * v, 16)]
                    pltpu.async_copy(h_hbm.at[idx],
                                     rows_v.at[buf, pl.ds(16 * v, 16)],
                                     sems[buf])

            for k in range(nbuf - 1):
                fire(k, k)

            def target(i, carry):
                gi = base + i
                buf = lax.rem(i, nbuf)
                nxt = lax.rem(i + nbuf - 1, nbuf)

                for k in range(nbuf):
                    @pl.when(jnp.logical_and(i + nbuf - 1 < _TB, nxt == k))
                    def _(k=k):
                        fire(i + nbuf - 1, k)

                idxs = [nei_ts[i, pl.ds(16 * v, 16)] for v in range(nvec)]
                pr = plsc.load_gather(pr_ts, [jnp.full((16,), gi, jnp.int32)])
                lgs = []
                for v in range(nvec):
                    pv = plsc.load_gather(pv_ts, [idxs[v]])
                    lg = pr + pv
                    lgs.append(jnp.where(lg >= 0.0, lg, 0.01 * lg))
                m = jnp.max(lgs[0])
                for v in range(1, nvec):
                    m = jnp.maximum(m, jnp.max(lgs[v]))
                exs = [jnp.exp(lg - m) for lg in lgs]
                ssum = jnp.sum(exs[0])
                for v in range(1, nvec):
                    ssum = ssum + jnp.sum(exs[v])
                denom = jnp.full((16,), ssum, jnp.float32)
                ws_all = [exs[v] / denom for v in range(nvec)]

                for k in range(nbuf):
                    @pl.when(buf == k)
                    def _(k=k):
                        pltpu.make_async_copy(
                            h_hbm.at[pl.ds(0, s_count)],
                            rows_v.at[k, pl.ds(0, s_count)], sems[k]).wait()

                iota = lax.iota(jnp.int32, 16)
                accs = [jnp.zeros((16,), jnp.float32)
                        for _ in range(_D // 16)]
                for s_ in range(s_count):
                    wv = ws_all[s_ // 16]
                    ws = jnp.full(
                        (16,),
                        jnp.sum(jnp.where(iota == (s_ % 16), wv, 0.0)),
                        jnp.float32)
                    for dc in range(_D // 16):
                        accs[dc] = accs[dc] + ws * rows_v[buf, s_,
                                                          pl.ds(16 * dc, 16)]
                for dc in range(_D // 16):
                    a = accs[dc]
                    e_buf[i, pl.ds(16 * dc, 16)] = jnp.where(
                        a > 0.0, a, jnp.exp(a) - 1.0)
                return carry

            lax.fori_loop(0, _TB, target, 0)
            pltpu.sync_copy(e_buf, e_hbm.at[pl.ds(base, _TB)])

        run_view(h1_hbm, nei0_ts, _S0, p0_ts, p2_ts, e0_hbm)
        run_view(h2_hbm, nei1_ts, _S1, p1_ts, p3_ts, e1_hbm)

    return body(h1, h2, nei0p, nei1p, P)


def _prep_body(h0_ref, e0_ref, e1_ref, fcw_ref, fcb_ref,
               h0n_ref, e0n_ref, e1n_ref, sp0_ref, sp1_ref):
    i = pl.program_id(0)

    def nrm(x):
        n = jnp.sqrt(jnp.sum(x * x, axis=1, keepdims=True))
        return x / (n + 1e-8)

    e0 = e0_ref[...]
    e1 = e1_ref[...]
    h0n_ref[...] = nrm(h0_ref[...])
    e0n_ref[...] = nrm(e0)
    e1n_ref[...] = nrm(e1)
    dn = (((1,), (1,)), ((), ()))
    fcw = fcw_ref[...]
    fcb = fcb_ref[...]
    t0 = jnp.tanh(lax.dot_general(e0, fcw, dn,
                                  preferred_element_type=jnp.float32) + fcb)
    t1 = jnp.tanh(lax.dot_general(e1, fcw, dn,
                                  preferred_element_type=jnp.float32) + fcb)

    @pl.when(i == 0)
    def _():
        sp0_ref[...] = jnp.zeros_like(sp0_ref)
        sp1_ref[...] = jnp.zeros_like(sp1_ref)

    sp0_ref[...] += jnp.sum(t0, axis=0, keepdims=True)
    sp1_ref[...] += jnp.sum(t1, axis=0, keepdims=True)


def _prep(h0, e0, e1, fc_W, fc_b2):
    B = 1000
    row = lambda i: (i, 0)
    fixed = lambda i: (0, 0)
    return pl.pallas_call(
        _prep_body,
        grid=(_N // B,),
        in_specs=[
            pl.BlockSpec((B, _D), row),
            pl.BlockSpec((B, _D), row),
            pl.BlockSpec((B, _D), row),
            pl.BlockSpec((_D, _D), fixed),
            pl.BlockSpec((1, _D), fixed),
        ],
        out_specs=[
            pl.BlockSpec((B, _D), row),
            pl.BlockSpec((B, _D), row),
            pl.BlockSpec((B, _D), row),
            pl.BlockSpec((1, _D), fixed),
            pl.BlockSpec((1, _D), fixed),
        ],
        out_shape=[
            jax.ShapeDtypeStruct((_N, _D), jnp.float32),
            jax.ShapeDtypeStruct((_N, _D), jnp.float32),
            jax.ShapeDtypeStruct((_N, _D), jnp.float32),
            jax.ShapeDtypeStruct((1, _D), jnp.float32),
            jax.ShapeDtypeStruct((1, _D), jnp.float32),
        ],
    )(h0, e0, e1, fc_W, fc_b2)


def _flash_body(h0n_ref, e0n_ref, e1n_ref, e0_ref, e1_ref,
                e0nf_ref, e1nf_ref, sp0_ref, sp1_ref, ai_ref,
                z_ref, la0_ref, la1_ref, la2_ref):
    i = pl.program_id(0)
    inv_tau = 1.0 / _TAU
    dn = (((1,), (1,)), ((), ()))
    h0n = h0n_ref[...]
    e0n = e0n_ref[...]
    e1n = e1n_ref[...]
    e0nf = e0nf_ref[...]
    e1nf = e1nf_ref[...]

    def ntx_part(rows, colsf, diag_rows):
        s = lax.dot_general(rows, colsf, dn,
                            preferred_element_type=jnp.float32) * inv_tau
        lse = jnp.log(jnp.sum(jnp.exp(s), axis=1, keepdims=True))
        d = jnp.sum(rows * diag_rows, axis=1, keepdims=True) * inv_tau
        return jnp.sum(lse - d)

    p0 = ntx_part(h0n, e0nf, e0n)
    p1 = ntx_part(h0n, e1nf, e1n)
    p2 = ntx_part(e0n, e1nf, e1n)

    @pl.when(i == 0)
    def _():
        la0_ref[...] = jnp.zeros_like(la0_ref)
        la1_ref[...] = jnp.zeros_like(la1_ref)
        la2_ref[...] = jnp.zeros_like(la2_ref)

    la0_ref[...] += jnp.full((1, _D), p0, jnp.float32)
    la1_ref[...] += jnp.full((1, _D), p1, jnp.float32)
    la2_ref[...] += jnp.full((1, _D), p2, jnp.float32)

    ai = ai_ref[...]
    b0 = jnp.sum(ai * sp0_ref[...]) * (1.0 / _N)
    b1 = jnp.sum(ai * sp1_ref[...]) * (1.0 / _N)
    t = jnp.exp(jnp.full((1, _D), b1 - b0, jnp.float32))
    beta0 = 1.0 / (1.0 + t)
    e0b = e0_ref[...]
    e1b = e1_ref[...]
    z_ref[...] = e1b + beta0 * (e0b - e1b)


def _flash(h0n, e0n, e1n, e0, e1, sp0s, sp1s, att_inter):
    B = 200
    row = lambda i: (i, 0)
    fixed = lambda i: (0, 0)
    return pl.pallas_call(
        _flash_body,
        grid=(_N // B,),
        in_specs=[
            pl.BlockSpec((B, _D), row),
            pl.BlockSpec((B, _D), row),
            pl.BlockSpec((B, _D), row),
            pl.BlockSpec((B, _D), row),
            pl.BlockSpec((B, _D), row),
            pl.BlockSpec((_N, _D), fixed),
            pl.BlockSpec((_N, _D), fixed),
            pl.BlockSpec((1, _D), fixed),
            pl.BlockSpec((1, _D), fixed),
            pl.BlockSpec((1, _D), fixed),
        ],
        out_specs=[
            pl.BlockSpec((B, _D), row),
            pl.BlockSpec((1, _D), fixed),
            pl.BlockSpec((1, _D), fixed),
            pl.BlockSpec((1, _D), fixed),
        ],
        out_shape=[
            jax.ShapeDtypeStruct((_N, _D), jnp.float32),
            jax.ShapeDtypeStruct((1, _D), jnp.float32),
            jax.ShapeDtypeStruct((1, _D), jnp.float32),
            jax.ShapeDtypeStruct((1, _D), jnp.float32),
        ],
    )(h0n, e0n, e1n, e0, e1, e0n, e1n, sp0s, sp1s, att_inter)


def kernel(h0, h1, h2, nei0, nei1, att0, att1, fc_W, fc_b, att_inter):
    pad = _NPAD - _N
    h0p = jnp.pad(h0, ((0, pad), (0, 0)))
    h1p = jnp.pad(h1, ((0, pad), (0, 0)))
    h2p = jnp.pad(h2, ((0, pad), (0, 0)))
    nei0p = jnp.pad(nei0, ((0, pad), (0, 0)))
    nei1p = jnp.pad(nei1, ((0, pad), (0, 0)))

    P = _proj(h0p, h1p, h2p, att0, att1)
    e0p, e1p = _sc_agg(h1, h2, nei0p, nei1p, P)
    e0 = e0p[:_N]
    e1 = e1p[:_N]
    h0n, e0n, e1n, sp0s, sp1s = _prep(h0, e0, e1, fc_W,
                                      fc_b.reshape(1, _D))
    z_mc, la0, la1, la2 = _flash(h0n, e0n, e1n, e0, e1, sp0s, sp1s,
                                 att_inter)
    loss_total = (la0[0, 0] + la1[0, 0] + _ALPHA * la2[0, 0]) / _N
    return (z_mc, loss_total)
